# baseline (speedup 1.0000x reference)
import functools

import jax
import jax.numpy as jnp
from jax import lax
from jax.experimental import pallas as pl
from jax.experimental.pallas import tpu as pltpu
from jax.experimental.pallas import tpu_sc as plsc

_LANES = 16
_NC = 2
_NS = 16
_NW = _NC * _NS
_C = 2048
_NBUF = 2


def _regrid_body(M, N, P, zf_hbm, idx_hbm, w_hbm, out_hbm,
                 zrow, idxv0, idxv1, wv0, wv1, outv,
                 isem0, isem1, osem0, osem1):
    isems = (isem0, isem1)
    osems = (osem0, osem1)
    idxvs = (idxv0, idxv1)
    wvs = (wv0, wv1)
    nchunks = N // _C
    wid = lax.axis_index("s") * _NC + lax.axis_index("c")

    def start_in(k, c):
        pltpu.async_copy(idx_hbm.at[pl.ds(c * _C * P, _C * P)], idxvs[k],
                         isems[k])
        pltpu.async_copy(w_hbm.at[pl.ds(c * _C * P, _C * P)], wvs[k],
                         isems[k])

    def wait_in(k):
        pltpu.make_async_copy(idx_hbm.at[pl.ds(0, _C * P)], idxvs[k],
                              isems[k]).wait()
        pltpu.make_async_copy(w_hbm.at[pl.ds(0, _C * P)], wvs[k],
                              isems[k]).wait()

    def wait_out(k):
        pltpu.make_async_copy(outv.at[k],
                              out_hbm.at[pl.ds(0, 2), pl.ds(0, _C)],
                              osems[k]).wait()

    maskhi = jnp.full((_LANES,), -65536, jnp.int32)
    one16 = jnp.full((_LANES,), 1, jnp.int32)
    half16 = jnp.full((_LANES,), 0x7FFF, jnp.int32)
    lane4 = lax.iota(jnp.int32, _LANES) * P

    b0 = 2 * wid

    def rne(x):
        odd = lax.bitwise_and(lax.shift_right_logical(x, 16), one16)
        return x + half16 + odd

    def stage(s, _):
        pltpu.sync_copy(zf_hbm.at[pl.ds(b0, 2), pl.ds(s * _C, _C)],
                        outv.at[0])

        @plsc.parallel_loop(0, _C // _LANES, unroll=4)
        def _pack(i):
            e = plsc.bitcast(outv[0, 0, pl.ds(i * _LANES, _LANES)],
                             jnp.int32)
            o = plsc.bitcast(outv[0, 1, pl.ds(i * _LANES, _LANES)],
                             jnp.int32)
            lo = lax.shift_right_logical(rne(e), 16)
            hi = lax.bitwise_and(rne(o), maskhi)
            zrow[pl.ds(s * _C + i * _LANES, _LANES)] = \
                lax.bitwise_or(lo, hi)
        return _

    lax.fori_loop(0, M // _C, stage, 0)

    for k in range(_NBUF):
        start_in(k, k)

    def pair_body(j, _):
        for k in range(_NBUF):
            c = _NBUF * j + k
            wait_in(k)

            @pl.when(j >= 1)
            def _wait():
                wait_out(k)

            @plsc.parallel_loop(0, _C // _LANES, unroll=4)
            def _inner(i):
                a0 = lane4 + i * (_LANES * P)
                a1 = a0 + 1
                a2 = a0 + 2
                a3 = a0 + 3
                i0 = plsc.load_gather(idxvs[k], [a0])
                i1 = plsc.load_gather(idxvs[k], [a1])
                i2 = plsc.load_gather(idxvs[k], [a2])
                i3 = plsc.load_gather(idxvs[k], [a3])
                w0 = plsc.load_gather(wvs[k], [a0])
                w1 = plsc.load_gather(wvs[k], [a1])
                w2 = plsc.load_gather(wvs[k], [a2])
                w3 = plsc.load_gather(wvs[k], [a3])
                g0 = plsc.load_gather(zrow, [i0])
                g1 = plsc.load_gather(zrow, [i1])
                g2 = plsc.load_gather(zrow, [i2])
                g3 = plsc.load_gather(zrow, [i3])
                acc0 = w0 * plsc.bitcast(lax.shift_left(g0, 16), jnp.float32)
                acc1 = w0 * plsc.bitcast(lax.bitwise_and(g0, maskhi),
                                         jnp.float32)
                acc0 = acc0 + w1 * plsc.bitcast(lax.shift_left(g1, 16),
                                                jnp.float32)
                acc1 = acc1 + w1 * plsc.bitcast(lax.bitwise_and(g1, maskhi),
                                                jnp.float32)
                acc0 = acc0 + w2 * plsc.bitcast(lax.shift_left(g2, 16),
                                                jnp.float32)
                acc1 = acc1 + w2 * plsc.bitcast(lax.bitwise_and(g2, maskhi),
                                                jnp.float32)
                acc0 = acc0 + w3 * plsc.bitcast(lax.shift_left(g3, 16),
                                                jnp.float32)
                acc1 = acc1 + w3 * plsc.bitcast(lax.bitwise_and(g3, maskhi),
                                                jnp.float32)
                outv[k, 0, pl.ds(i * _LANES, _LANES)] = acc0
                outv[k, 1, pl.ds(i * _LANES, _LANES)] = acc1

            pltpu.async_copy(outv.at[k],
                             out_hbm.at[pl.ds(b0, 2), pl.ds(c * _C, _C)],
                             osems[k])

            @pl.when(j < nchunks // _NBUF - 1)
            def _next():
                start_in(k, c + _NBUF)
        return _

    lax.fori_loop(0, nchunks // _NBUF, pair_body, 0)
    for k in range(_NBUF):
        wait_out(k)


def _regrid(zf, idxf, wf, P):
    B, M = zf.shape
    N = idxf.shape[0] // P
    mesh = plsc.VectorSubcoreMesh(
        core_axis_name="c", subcore_axis_name="s",
        num_cores=_NC, num_subcores=_NS)
    fn = pl.kernel(
        functools.partial(_regrid_body, M, N, P),
        out_type=jax.ShapeDtypeStruct((B, N), jnp.float32),
        mesh=mesh,
        compiler_params=pltpu.CompilerParams(needs_layout_passes=False),
        scratch_types=[
            pltpu.VMEM((M,), jnp.int32),
            pltpu.VMEM((_C * P,), jnp.int32),
            pltpu.VMEM((_C * P,), jnp.int32),
            pltpu.VMEM((_C * P,), jnp.float32),
            pltpu.VMEM((_C * P,), jnp.float32),
            pltpu.VMEM((_NBUF, 2, _C), jnp.float32),
            pltpu.SemaphoreType.DMA,
            pltpu.SemaphoreType.DMA,
            pltpu.SemaphoreType.DMA,
            pltpu.SemaphoreType.DMA,
        ],
    )
    return fn(zf, idxf, wf)


def kernel(z, index, weight):
    batch = z.shape[:-1]
    M = z.shape[-1]
    out_shape = index.shape[:-1]
    P = index.shape[-1]
    zf = z.reshape(-1, M)
    idxf = index.reshape(-1)
    wf = weight.reshape(-1)
    out = _regrid(zf, idxf, wf, P)
    return out.reshape(batch + out_shape)


# speedup vs baseline: 1.8372x; 1.8372x over previous
import functools

import jax
import jax.numpy as jnp
from jax import lax
from jax.experimental import pallas as pl
from jax.experimental.pallas import tpu as pltpu
from jax.experimental.pallas import tpu_sc as plsc

_LANES = 16
_NC = 2
_NS = 16
_NW = _NC * _NS
_C = 4096
_NBUF = 2


def _regrid_body(M, N, zf_hbm, idxp_hbm, wp_hbm, out_hbm,
                 zrow, idxv, wv, outv,
                 isem0, isem1, osem0, osem1):
    isems = (isem0, isem1)
    osems = (osem0, osem1)
    nchunks = N // _C
    wid = lax.axis_index("s") * _NC + lax.axis_index("c")

    def start_in(k, c):
        pltpu.async_copy(idxp_hbm.at[:, pl.ds(c * _C, _C)], idxv.at[k],
                         isems[k])
        pltpu.async_copy(wp_hbm.at[:, pl.ds(c * _C, _C)], wv.at[k], isems[k])

    def wait_in(k):
        pltpu.make_async_copy(idxp_hbm.at[:, pl.ds(0, _C)], idxv.at[k],
                              isems[k]).wait()
        pltpu.make_async_copy(wp_hbm.at[:, pl.ds(0, _C)], wv.at[k],
                              isems[k]).wait()

    def wait_out(k):
        pltpu.make_async_copy(outv.at[k],
                              out_hbm.at[pl.ds(0, 2), pl.ds(0, _C)],
                              osems[k]).wait()

    mask16 = jnp.full((_LANES,), 0xFFFF, jnp.int32)
    maskhi = jnp.full((_LANES,), -65536, jnp.int32)
    one16 = jnp.full((_LANES,), 1, jnp.int32)
    half16 = jnp.full((_LANES,), 0x7FFF, jnp.int32)

    b0 = 2 * wid

    def rne(x):
        odd = lax.bitwise_and(lax.shift_right_logical(x, 16), one16)
        return x + half16 + odd

    def stage(s, _):
        pltpu.sync_copy(zf_hbm.at[pl.ds(b0, 2), pl.ds(s * _C, _C)],
                        outv.at[0])

        @plsc.parallel_loop(0, _C // _LANES, unroll=4)
        def _pack(i):
            e = plsc.bitcast(outv[0, 0, pl.ds(i * _LANES, _LANES)],
                             jnp.int32)
            o = plsc.bitcast(outv[0, 1, pl.ds(i * _LANES, _LANES)],
                             jnp.int32)
            lo = lax.shift_right_logical(rne(e), 16)
            hi = lax.bitwise_and(rne(o), maskhi)
            zrow[pl.ds(s * _C + i * _LANES, _LANES)] = \
                lax.bitwise_or(lo, hi)
        return _

    lax.fori_loop(0, M // _C, stage, 0)

    for k in range(_NBUF):
        start_in(k, k)

    def pair_body(j, _):
        for k in range(_NBUF):
            c = _NBUF * j + k
            wait_in(k)

            @pl.when(j >= 1)
            def _wait():
                wait_out(k)

            @plsc.parallel_loop(0, _C // _LANES, unroll=4)
            def _inner(i):
                v01 = idxv[k, 0, pl.ds(i * _LANES, _LANES)]
                v23 = idxv[k, 1, pl.ds(i * _LANES, _LANES)]
                w01 = wv[k, 0, pl.ds(i * _LANES, _LANES)]
                w23 = wv[k, 1, pl.ds(i * _LANES, _LANES)]
                i0 = lax.bitwise_and(v01, mask16)
                i1 = lax.shift_right_logical(v01, 16)
                i2 = lax.bitwise_and(v23, mask16)
                i3 = lax.shift_right_logical(v23, 16)
                w0 = plsc.bitcast(lax.shift_left(w01, 16), jnp.float32)
                w1 = plsc.bitcast(lax.bitwise_and(w01, maskhi), jnp.float32)
                w2 = plsc.bitcast(lax.shift_left(w23, 16), jnp.float32)
                w3 = plsc.bitcast(lax.bitwise_and(w23, maskhi), jnp.float32)
                g0 = plsc.load_gather(zrow, [i0])
                g1 = plsc.load_gather(zrow, [i1])
                g2 = plsc.load_gather(zrow, [i2])
                g3 = plsc.load_gather(zrow, [i3])
                acc0 = w0 * plsc.bitcast(lax.shift_left(g0, 16), jnp.float32)
                acc1 = w0 * plsc.bitcast(lax.bitwise_and(g0, maskhi),
                                         jnp.float32)
                acc0 = acc0 + w1 * plsc.bitcast(lax.shift_left(g1, 16),
                                                jnp.float32)
                acc1 = acc1 + w1 * plsc.bitcast(lax.bitwise_and(g1, maskhi),
                                                jnp.float32)
                acc0 = acc0 + w2 * plsc.bitcast(lax.shift_left(g2, 16),
                                                jnp.float32)
                acc1 = acc1 + w2 * plsc.bitcast(lax.bitwise_and(g2, maskhi),
                                                jnp.float32)
                acc0 = acc0 + w3 * plsc.bitcast(lax.shift_left(g3, 16),
                                                jnp.float32)
                acc1 = acc1 + w3 * plsc.bitcast(lax.bitwise_and(g3, maskhi),
                                                jnp.float32)
                outv[k, 0, pl.ds(i * _LANES, _LANES)] = acc0
                outv[k, 1, pl.ds(i * _LANES, _LANES)] = acc1

            pltpu.async_copy(outv.at[k],
                             out_hbm.at[pl.ds(b0, 2), pl.ds(c * _C, _C)],
                             osems[k])

            @pl.when(j < nchunks // _NBUF - 1)
            def _next():
                start_in(k, c + _NBUF)
        return _

    lax.fori_loop(0, nchunks // _NBUF, pair_body, 0)
    for k in range(_NBUF):
        wait_out(k)


def _regrid(zf, idxp, wp):
    B, M = zf.shape
    _, N = idxp.shape
    mesh = plsc.VectorSubcoreMesh(
        core_axis_name="c", subcore_axis_name="s",
        num_cores=_NC, num_subcores=_NS)
    fn = pl.kernel(
        functools.partial(_regrid_body, M, N),
        out_type=jax.ShapeDtypeStruct((B, N), jnp.float32),
        mesh=mesh,
        compiler_params=pltpu.CompilerParams(needs_layout_passes=False, use_tc_tiling_on_sc=True),
        scratch_types=[
            pltpu.VMEM((M,), jnp.int32),
            pltpu.VMEM((_NBUF, 2, _C), jnp.int32),
            pltpu.VMEM((_NBUF, 2, _C), jnp.int32),
            pltpu.VMEM((_NBUF, 2, _C), jnp.float32),
            pltpu.SemaphoreType.DMA,
            pltpu.SemaphoreType.DMA,
            pltpu.SemaphoreType.DMA,
            pltpu.SemaphoreType.DMA,
        ],
    )
    return fn(zf, idxp, wp)


def kernel(z, index, weight):
    batch = z.shape[:-1]
    M = z.shape[-1]
    out_shape = index.shape[:-1]
    P = index.shape[-1]
    zf = z.reshape(-1, M)
    idx = index.reshape(-1, P)
    idxp = lax.bitwise_or(
        idx[:, 0::2].T, lax.shift_left(idx[:, 1::2].T, 16))
    w16 = lax.bitcast_convert_type(
        weight.reshape(-1, P).astype(jnp.bfloat16), jnp.uint16
    ).astype(jnp.int32)
    wp = lax.bitwise_or(w16[:, 0::2].T, lax.shift_left(w16[:, 1::2].T, 16))
    out = _regrid(zf, idxp, wp)
    return out.reshape(batch + out_shape)


# speedup vs baseline: 2.2323x; 1.2150x over previous
import functools

import jax
import jax.numpy as jnp
from jax import lax
from jax.experimental import pallas as pl
from jax.experimental.pallas import tpu as pltpu
from jax.experimental.pallas import tpu_sc as plsc

_LANES = 16
_NC = 2
_NS = 16
_NW = _NC * _NS
_C = 4096
_NBUF = 2


def _regrid_body(M, N, zf_hbm, ilo_hbm, ihi_hbm, wlo_hbm, whi_hbm, out_hbm,
                 zrow, idxv, wv, outv,
                 isem0, isem1, osem0, osem1):
    isems = (isem0, isem1)
    osems = (osem0, osem1)
    nchunks = N // _C
    wid = lax.axis_index("s") * _NC + lax.axis_index("c")

    def start_in(k, c):
        sl = pl.ds(c * _C, _C)
        pltpu.async_copy(ilo_hbm.at[sl], idxv.at[k, 0], isems[k])
        pltpu.async_copy(ihi_hbm.at[sl], idxv.at[k, 1], isems[k])
        pltpu.async_copy(wlo_hbm.at[sl], wv.at[k, 0], isems[k])
        pltpu.async_copy(whi_hbm.at[sl], wv.at[k, 1], isems[k])

    def wait_in(k):
        sl = pl.ds(0, _C)
        pltpu.make_async_copy(ilo_hbm.at[sl], idxv.at[k, 0],
                              isems[k]).wait()
        pltpu.make_async_copy(ihi_hbm.at[sl], idxv.at[k, 1],
                              isems[k]).wait()
        pltpu.make_async_copy(wlo_hbm.at[sl], wv.at[k, 0],
                              isems[k]).wait()
        pltpu.make_async_copy(whi_hbm.at[sl], wv.at[k, 1],
                              isems[k]).wait()

    def wait_out(k):
        pltpu.make_async_copy(outv.at[k],
                              out_hbm.at[pl.ds(0, 2), pl.ds(0, _C)],
                              osems[k]).wait()

    mask16 = jnp.full((_LANES,), 0xFFFF, jnp.int32)
    maskhi = jnp.full((_LANES,), -65536, jnp.int32)
    one16 = jnp.full((_LANES,), 1, jnp.int32)
    half16 = jnp.full((_LANES,), 0x7FFF, jnp.int32)

    b0 = 2 * wid

    def rne(x):
        odd = lax.bitwise_and(lax.shift_right_logical(x, 16), one16)
        return x + half16 + odd

    def stage(s, _):
        pltpu.sync_copy(zf_hbm.at[pl.ds(b0, 2), pl.ds(s * _C, _C)],
                        outv.at[0])

        @plsc.parallel_loop(0, _C // _LANES, unroll=4)
        def _pack(i):
            e = plsc.bitcast(outv[0, 0, pl.ds(i * _LANES, _LANES)],
                             jnp.int32)
            o = plsc.bitcast(outv[0, 1, pl.ds(i * _LANES, _LANES)],
                             jnp.int32)
            lo = lax.shift_right_logical(rne(e), 16)
            hi = lax.bitwise_and(rne(o), maskhi)
            zrow[pl.ds(s * _C + i * _LANES, _LANES)] = \
                lax.bitwise_or(lo, hi)
        return _

    lax.fori_loop(0, M // _C, stage, 0)

    for k in range(_NBUF):
        start_in(k, k)

    def pair_body(j, _):
        for k in range(_NBUF):
            c = _NBUF * j + k
            wait_in(k)

            @pl.when(j >= 1)
            def _wait():
                wait_out(k)

            @plsc.parallel_loop(0, _C // _LANES, unroll=4)
            def _inner(i):
                v01 = idxv[k, 0, pl.ds(i * _LANES, _LANES)]
                v23 = idxv[k, 1, pl.ds(i * _LANES, _LANES)]
                w01 = wv[k, 0, pl.ds(i * _LANES, _LANES)]
                w23 = wv[k, 1, pl.ds(i * _LANES, _LANES)]
                i0 = lax.bitwise_and(v01, mask16)
                i1 = lax.shift_right_logical(v01, 16)
                i2 = lax.bitwise_and(v23, mask16)
                i3 = lax.shift_right_logical(v23, 16)
                w0 = plsc.bitcast(lax.shift_left(w01, 16), jnp.float32)
                w1 = plsc.bitcast(lax.bitwise_and(w01, maskhi), jnp.float32)
                w2 = plsc.bitcast(lax.shift_left(w23, 16), jnp.float32)
                w3 = plsc.bitcast(lax.bitwise_and(w23, maskhi), jnp.float32)
                g0 = plsc.load_gather(zrow, [i0])
                g1 = plsc.load_gather(zrow, [i1])
                g2 = plsc.load_gather(zrow, [i2])
                g3 = plsc.load_gather(zrow, [i3])
                acc0 = w0 * plsc.bitcast(lax.shift_left(g0, 16), jnp.float32)
                acc1 = w0 * plsc.bitcast(lax.bitwise_and(g0, maskhi),
                                         jnp.float32)
                acc0 = acc0 + w1 * plsc.bitcast(lax.shift_left(g1, 16),
                                                jnp.float32)
                acc1 = acc1 + w1 * plsc.bitcast(lax.bitwise_and(g1, maskhi),
                                                jnp.float32)
                acc0 = acc0 + w2 * plsc.bitcast(lax.shift_left(g2, 16),
                                                jnp.float32)
                acc1 = acc1 + w2 * plsc.bitcast(lax.bitwise_and(g2, maskhi),
                                                jnp.float32)
                acc0 = acc0 + w3 * plsc.bitcast(lax.shift_left(g3, 16),
                                                jnp.float32)
                acc1 = acc1 + w3 * plsc.bitcast(lax.bitwise_and(g3, maskhi),
                                                jnp.float32)
                outv[k, 0, pl.ds(i * _LANES, _LANES)] = acc0
                outv[k, 1, pl.ds(i * _LANES, _LANES)] = acc1

            pltpu.async_copy(outv.at[k],
                             out_hbm.at[pl.ds(b0, 2), pl.ds(c * _C, _C)],
                             osems[k])

            @pl.when(j < nchunks // _NBUF - 1)
            def _next():
                start_in(k, c + _NBUF)
        return _

    lax.fori_loop(0, nchunks // _NBUF, pair_body, 0)
    for k in range(_NBUF):
        wait_out(k)


def _regrid(zf, ilo, ihi, wlo, whi):
    B, M = zf.shape
    N = ilo.shape[0]
    mesh = plsc.VectorSubcoreMesh(
        core_axis_name="c", subcore_axis_name="s",
        num_cores=_NC, num_subcores=_NS)
    fn = pl.kernel(
        functools.partial(_regrid_body, M, N),
        out_type=jax.ShapeDtypeStruct((B, N), jnp.float32),
        mesh=mesh,
        compiler_params=pltpu.CompilerParams(needs_layout_passes=False),
        scratch_types=[
            pltpu.VMEM((M,), jnp.int32),
            pltpu.VMEM((_NBUF, 2, _C), jnp.int32),
            pltpu.VMEM((_NBUF, 2, _C), jnp.int32),
            pltpu.VMEM((_NBUF, 2, _C), jnp.float32),
            pltpu.SemaphoreType.DMA,
            pltpu.SemaphoreType.DMA,
            pltpu.SemaphoreType.DMA,
            pltpu.SemaphoreType.DMA,
        ],
    )
    return fn(zf, ilo, ihi, wlo, whi)


def kernel(z, index, weight):
    batch = z.shape[:-1]
    M = z.shape[-1]
    out_shape = index.shape[:-1]
    P = index.shape[-1]
    zf = z.reshape(-1, M)
    idx = index.reshape(-1, P)
    ilo = lax.bitwise_or(idx[:, 0], lax.shift_left(idx[:, 1], 16))
    ihi = lax.bitwise_or(idx[:, 2], lax.shift_left(idx[:, 3], 16))
    w16 = lax.bitcast_convert_type(
        weight.reshape(-1, P).astype(jnp.bfloat16), jnp.uint16
    ).astype(jnp.int32)
    wlo = lax.bitwise_or(w16[:, 0], lax.shift_left(w16[:, 1], 16))
    whi = lax.bitwise_or(w16[:, 2], lax.shift_left(w16[:, 3], 16))
    out = _regrid(zf, ilo, ihi, wlo, whi)
    return out.reshape(batch + out_shape)
